# baseline (speedup 1.0000x reference)
import functools

import jax
import jax.numpy as jnp
from jax import lax
from jax.experimental import pallas as pl
from jax.experimental.pallas import tpu as pltpu
from jax.experimental.pallas import tpu_sc as plsc

_R = 80
_NT = 625
_NW = 32
_TPW = 20


def _prep_kernel(epsx_ref, g_ref, asm_ref, temb_ref, *, n_rows, dh):
    colsum = jnp.sum(epsx_ref[...], axis=0, keepdims=True)
    ph = jax.lax.broadcasted_iota(jnp.int32, (1, 120), 1) % 3
    lane = jax.lax.broadcasted_iota(jnp.int32, (1, dh), 1)
    m = [jnp.sum(jnp.where(ph == c, colsum, 0.0)) * (1.0 / n_rows)
         for c in range(3)]
    msplat = jnp.where(
        lane < 16, m[0],
        jnp.where(lane < 32, m[1], jnp.where(lane < 48, m[2], 0.0)))

    g = jnp.pad(g_ref[...], ((0, 0), (0, dh - 100)))
    alpha = jnp.sqrt(1.0 / (1.0 + jnp.exp(g)))
    sigma = jnp.sqrt(1.0 / (1.0 + jnp.exp(-g)))
    rowi = jax.lax.broadcasted_iota(jnp.int32, (8, dh), 0)
    asm_ref[...] = jnp.where(
        rowi == 0, alpha, jnp.where(rowi == 1, sigma,
                                    jnp.where(rowi == 2, msplat, 0.0)))

    row = jax.lax.broadcasted_iota(jnp.int32, (100, dh), 0
                                   ).astype(jnp.float32)
    col = jax.lax.broadcasted_iota(jnp.int32, (100, dh), 1
                                   ).astype(jnp.float32)
    k = jnp.where(col < 64.0, col, col - 64.0)
    freqs = jnp.exp(k * (-jnp.log(10000.0) / 64.0))
    xf = row * freqs
    temb_ref[...] = jnp.where(col < 64.0, jnp.cos(xf), jnp.sin(xf))


def _sc_body(pos_hbm, h_hbm, eps_hbm, b_hbm, asm_hbm, temb_hbm, out_hbm,
             pos_v, h_v, eps_v, b_v, t_v, o_v, asm_v, sem):
    wid = lax.axis_index("s") * 2 + lax.axis_index("c")
    pltpu.sync_copy(asm_hbm, asm_v)
    iota = lax.iota(jnp.int32, 16)
    zero16 = jnp.zeros((16,), jnp.int32)
    one16 = jnp.full((16,), 1, jnp.int32)
    two16 = jnp.full((16,), 2, jnp.int32)

    def tile_step(i, carry):
        t = wid + i * _NW

        @pl.when(t < _NT)
        def _():
            base = t * _R
            pltpu.sync_copy(pos_hbm.at[pl.ds(base, _R), :], pos_v)
            pltpu.sync_copy(h_hbm.at[pl.ds(base, _R), :], h_v)
            pltpu.sync_copy(eps_hbm.at[pl.ds(base, _R), :], eps_v)
            pltpu.sync_copy(b_hbm.at[pl.ds(base, _R)], b_v)
            pltpu.async_copy(temb_hbm.at[b_v], t_v, sem).wait()

            for grp in range(_R // 16):
                ridx = iota + (16 * grp)
                b_vec = b_v[pl.ds(16 * grp, 16)]
                a_vec = plsc.load_gather(asm_v, [zero16, b_vec])
                s_vec = plsc.load_gather(asm_v, [one16, b_vec])
                for c in range(3):
                    cc = jnp.full((16,), c, jnp.int32)
                    e = plsc.load_gather(eps_v, [ridx, cc])
                    x = plsc.load_gather(pos_v, [ridx, cc])
                    mc = plsc.load_gather(
                        asm_v, [two16, jnp.full((16,), 16 * c, jnp.int32)])
                    plsc.store_scatter(o_v, [ridx, cc],
                                       a_vec * x + s_vec * (e - mc))

                def zcol(c, carry2):
                    cc = zero16 + c
                    e = plsc.load_gather(eps_v, [ridx, cc])
                    x = plsc.load_gather(h_v, [ridx, cc - 3])
                    plsc.store_scatter(o_v, [ridx, cc],
                                       a_vec * x + s_vec * e)
                    return carry2

                lax.fori_loop(3, 131, zcol, 0, unroll=8)

                def tcol(j, carry2):
                    jj = zero16 + j
                    tv = plsc.load_gather(t_v, [ridx, jj])
                    plsc.store_scatter(o_v, [ridx, jj + 131], tv)
                    return carry2

                lax.fori_loop(0, 128, tcol, 0, unroll=8)

            pltpu.sync_copy(o_v, out_hbm.at[pl.ds(base, _R), :])

        return carry

    lax.fori_loop(0, _TPW, tile_step, 0)


def _sc_main(pos, h, eps, batch, asm, temb, n, d_out):
    mesh = plsc.VectorSubcoreMesh(core_axis_name="c", subcore_axis_name="s")
    return pl.kernel(
        _sc_body,
        out_type=jax.ShapeDtypeStruct((n, d_out), jnp.float32),
        mesh=mesh,
        scratch_types=[
            pltpu.VMEM((_R, 3), jnp.float32),
            pltpu.VMEM((_R, 128), jnp.float32),
            pltpu.VMEM((_R, 131), jnp.float32),
            pltpu.VMEM((_R,), jnp.int32),
            pltpu.VMEM((_R, 128), jnp.float32),
            pltpu.VMEM((_R, 259), jnp.float32),
            pltpu.VMEM((8, 128), jnp.float32),
            pltpu.SemaphoreType.DMA,
        ],
        compiler_params=pltpu.CompilerParams(needs_layout_passes=False),
    )(pos, h, eps, batch, asm, temb)


def kernel(pos, h, batch, eps, gamma):
    n, dh = h.shape
    d_out = 3 + dh + 128

    epsx = jax.lax.slice(eps, (0, 0), (n, 3)).reshape(1250, 120)
    g100 = gamma[:100].reshape(1, 100)
    prep = functools.partial(_prep_kernel, n_rows=n, dh=dh)
    asm, temb = pl.pallas_call(
        prep,
        grid=(1,),
        in_specs=[pl.BlockSpec((1250, 120), lambda i: (0, 0)),
                  pl.BlockSpec((1, 100), lambda i: (0, 0))],
        out_specs=[pl.BlockSpec((8, dh), lambda i: (0, 0)),
                   pl.BlockSpec((100, dh), lambda i: (0, 0))],
        out_shape=[jax.ShapeDtypeStruct((8, dh), jnp.float32),
                   jax.ShapeDtypeStruct((100, dh), jnp.float32)],
    )(epsx, g100)

    return _sc_main(pos, h, eps, batch, asm, temb, n, d_out)


# speedup vs baseline: 4.6026x; 4.6026x over previous
import functools

import jax
import jax.numpy as jnp
from jax.experimental import pallas as pl
from jax.experimental.pallas import tpu as pltpu


def _sum_kernel(eps_ref, out_ref):
    i = pl.program_id(0)

    @pl.when(i == 0)
    def _():
        out_ref[...] = jnp.zeros_like(out_ref)

    out_ref[...] += jnp.sum(eps_ref[...], axis=0, keepdims=True)


def _main_kernel(pos_ref, ss_ref, h_ref, eps_ref, g_ref, part_ref, out_ref,
                 tab_ref, mean_ref, *, n_rows, dh, de, bm):
    @pl.when(pl.program_id(0) == 0)
    def _():
        p = part_ref[...]
        ph = jax.lax.broadcasted_iota(jnp.int32, p.shape, 1) % 3
        lane = jax.lax.broadcasted_iota(jnp.int32, (1, de), 1)
        mp = jnp.zeros((1, de), jnp.float32)
        for c in range(3):
            m_c = jnp.sum(jnp.where(ph == c, p, 0.0)) * (1.0 / n_rows)
            mp = jnp.where(lane == c, m_c, mp)
        mean_ref[...] = mp

        g = g_ref[...]
        alpha = jnp.sqrt(1.0 / (1.0 + jnp.exp(g)))
        sigma = jnp.sqrt(1.0 / (1.0 + jnp.exp(-g)))
        tab_ref[:, 0:dh] = jnp.broadcast_to(alpha, (100, dh)
                                            ).astype(jnp.bfloat16)
        tab_ref[:, dh:2 * dh] = jnp.broadcast_to(sigma, (100, dh)
                                                 ).astype(jnp.bfloat16)
        row = jax.lax.broadcasted_iota(jnp.int32, (100, dh), 0
                                       ).astype(jnp.float32)
        col = jax.lax.broadcasted_iota(jnp.int32, (100, dh), 1
                                       ).astype(jnp.float32)
        k = jnp.where(col < 64.0, col, col - 64.0)
        freqs = jnp.exp(k * (-jnp.log(10000.0) / 64.0))
        xf = row * freqs
        temb = jnp.where(col < 64.0, jnp.cos(xf), jnp.sin(xf))
        tab_ref[:, 2 * dh:] = pltpu.roll(temb, 3, 1).astype(jnp.bfloat16)

    row_g = (pl.program_id(0) * bm
             + jax.lax.broadcasted_iota(jnp.int32, (bm, 1), 0))
    onehot = ((row_g >= ss_ref[0:1, :]) & (row_g < ss_ref[1:2, :])
              ).astype(jnp.bfloat16)
    r = jax.lax.dot_general(
        onehot, tab_ref[...], (((1,), (0,)), ((), ())),
        preferred_element_type=jnp.float32)
    a = r[:, 0:dh]
    s = r[:, dh:2 * dh]
    trot = r[:, 2 * dh:]

    lane = jax.lax.broadcasted_iota(jnp.int32, (1, dh), 1)
    hs = pltpu.roll(h_ref[...], 3, 1)
    pospad = jnp.pad(pos_ref[...], ((0, 0), (0, dh - 3)))
    xh0 = jnp.where(lane < 3, pospad, hs)
    epsm = eps_ref[...] - mean_ref[...]
    out_ref[:, 0:dh] = a * xh0 + s * epsm[:, 0:dh]
    z1 = a[:, 0:3] * hs[:, 0:3] + s[:, 0:3] * epsm[:, dh:de]
    z1pad = jnp.pad(z1, ((0, 0), (0, dh - 3)))
    out_ref[:, dh:2 * dh] = jnp.where(lane < 3, z1pad, trot)
    out_ref[:, 2 * dh:] = trot[:, 0:3]


def kernel(pos, h, batch, eps, gamma):
    n, dh = h.shape
    de = eps.shape[1]
    d_out = 3 + dh + 128

    epsx = jax.lax.slice(eps, (0, 0), (n, 3)).reshape(1250, 120)
    partials = pl.pallas_call(
        _sum_kernel,
        grid=(1,),
        in_specs=[pl.BlockSpec((1250, 120), lambda i: (0, 0))],
        out_specs=pl.BlockSpec((1, 120), lambda i: (0, 0)),
        out_shape=jax.ShapeDtypeStruct((1, 120), jnp.float32),
    )(epsx)

    bm = 5000
    g100 = gamma[:100].reshape(100, 1)
    ss = jnp.searchsorted(batch, jnp.arange(100, dtype=batch.dtype)
                          ).astype(jnp.int32)
    ssb = jnp.stack([ss, jnp.concatenate([ss[1:], jnp.array([n], jnp.int32)])])
    body = functools.partial(_main_kernel, n_rows=n, dh=dh, de=de, bm=bm)
    out = pl.pallas_call(
        body,
        grid=(n // bm,),
        in_specs=[
            pl.BlockSpec((bm, 3), lambda i: (i, 0)),
            pl.BlockSpec((2, 100), lambda i: (0, 0)),
            pl.BlockSpec((bm, dh), lambda i: (i, 0)),
            pl.BlockSpec((bm, de), lambda i: (i, 0)),
            pl.BlockSpec((100, 1), lambda i: (0, 0)),
            pl.BlockSpec((1, 120), lambda i: (0, 0)),
        ],
        out_specs=pl.BlockSpec((bm, d_out), lambda i: (i, 0)),
        out_shape=jax.ShapeDtypeStruct((n, d_out), jnp.float32),
        scratch_shapes=[pltpu.VMEM((100, 3 * dh), jnp.bfloat16),
                        pltpu.VMEM((1, de), jnp.float32)],
        compiler_params=pltpu.CompilerParams(
            dimension_semantics=("parallel",)),
    )(pos, ssb, h, eps, g100, partials)
    return out


# speedup vs baseline: 4.6100x; 1.0016x over previous
import functools

import jax
import jax.numpy as jnp
from jax.experimental import pallas as pl
from jax.experimental.pallas import tpu as pltpu


def _sum_kernel(eps_ref, out_ref):
    i = pl.program_id(0)

    @pl.when(i == 0)
    def _():
        out_ref[...] = jnp.zeros_like(out_ref)

    out_ref[...] += jnp.sum(eps_ref[...], axis=0, keepdims=True)


def _main_kernel(pos_ref, ss_ref, h_ref, eps_ref, g_ref, part_ref, out_ref,
                 tab_ref, mean_ref, *, n_rows, dh, de, bm):
    @pl.when(pl.program_id(0) == 0)
    def _():
        p = part_ref[...]
        ph = jax.lax.broadcasted_iota(jnp.int32, p.shape, 1) % 3
        lane = jax.lax.broadcasted_iota(jnp.int32, (1, de), 1)
        mp = jnp.zeros((1, de), jnp.float32)
        for c in range(3):
            m_c = jnp.sum(jnp.where(ph == c, p, 0.0)) * (1.0 / n_rows)
            mp = jnp.where(lane == c, m_c, mp)
        mean_ref[...] = mp

        g = g_ref[...]
        alpha = jnp.sqrt(1.0 / (1.0 + jnp.exp(g)))
        sigma = jnp.sqrt(1.0 / (1.0 + jnp.exp(-g)))
        tab_ref[:, 0:dh] = jnp.broadcast_to(alpha, (100, dh)
                                            ).astype(jnp.bfloat16)
        tab_ref[:, dh:2 * dh] = jnp.broadcast_to(sigma, (100, dh)
                                                 ).astype(jnp.bfloat16)
        row = jax.lax.broadcasted_iota(jnp.int32, (100, dh), 0
                                       ).astype(jnp.float32)
        col = jax.lax.broadcasted_iota(jnp.int32, (100, dh), 1
                                       ).astype(jnp.float32)
        k = jnp.where(col < 64.0, col, col - 64.0)
        freqs = jnp.exp(k * (-jnp.log(10000.0) / 64.0))
        xf = row * freqs
        temb = jnp.where(col < 64.0, jnp.cos(xf), jnp.sin(xf))
        tab_ref[:, 2 * dh:] = pltpu.roll(temb, 3, 1).astype(jnp.bfloat16)

    row_g = (pl.program_id(0) * bm
             + jax.lax.broadcasted_iota(jnp.int32, (bm, 1), 0))
    onehot = ((row_g >= ss_ref[0:1, :]) & (row_g < ss_ref[1:2, :])
              ).astype(jnp.bfloat16)
    r = jax.lax.dot_general(
        onehot, tab_ref[...], (((1,), (0,)), ((), ())),
        preferred_element_type=jnp.float32)
    a = r[:, 0:dh]
    s = r[:, dh:2 * dh]
    trot = r[:, 2 * dh:]

    lane = jax.lax.broadcasted_iota(jnp.int32, (1, dh), 1)
    hs = pltpu.roll(h_ref[...], 3, 1)
    pospad = jnp.pad(pos_ref[...], ((0, 0), (0, dh - 3)))
    xh0 = jnp.where(lane < 3, pospad, hs)
    epsm0 = eps_ref[:, 0:dh] - mean_ref[:, 0:dh]
    out_ref[:, 0:dh] = a * xh0 + s * epsm0
    z1 = a[:, 0:3] * hs[:, 0:3] + s[:, 0:3] * eps_ref[:, dh:de]
    z1pad = jnp.pad(z1, ((0, 0), (0, dh - 3)))
    out_ref[:, dh:2 * dh] = jnp.where(lane < 3, z1pad, trot)
    out_ref[:, 2 * dh:] = trot[:, 0:3]


def kernel(pos, h, batch, eps, gamma):
    n, dh = h.shape
    de = eps.shape[1]
    d_out = 3 + dh + 128

    epsx = jax.lax.slice(eps, (0, 0), (n, 3)).reshape(1250, 120)
    partials = pl.pallas_call(
        _sum_kernel,
        grid=(1,),
        in_specs=[pl.BlockSpec((1250, 120), lambda i: (0, 0))],
        out_specs=pl.BlockSpec((1, 120), lambda i: (0, 0)),
        out_shape=jax.ShapeDtypeStruct((1, 120), jnp.float32),
    )(epsx)

    bm = 5000
    g100 = gamma[:100].reshape(100, 1)
    ss = jnp.searchsorted(batch, jnp.arange(100, dtype=batch.dtype)
                          ).astype(jnp.int32)
    ssb = jnp.stack([ss, jnp.concatenate([ss[1:], jnp.array([n], jnp.int32)])])
    body = functools.partial(_main_kernel, n_rows=n, dh=dh, de=de, bm=bm)
    out = pl.pallas_call(
        body,
        grid=(n // bm,),
        in_specs=[
            pl.BlockSpec((bm, 3), lambda i: (i, 0)),
            pl.BlockSpec((2, 100), lambda i: (0, 0)),
            pl.BlockSpec((bm, dh), lambda i: (i, 0)),
            pl.BlockSpec((bm, de), lambda i: (i, 0)),
            pl.BlockSpec((100, 1), lambda i: (0, 0)),
            pl.BlockSpec((1, 120), lambda i: (0, 0)),
        ],
        out_specs=pl.BlockSpec((bm, d_out), lambda i: (i, 0)),
        out_shape=jax.ShapeDtypeStruct((n, d_out), jnp.float32),
        scratch_shapes=[pltpu.VMEM((100, 3 * dh), jnp.bfloat16),
                        pltpu.VMEM((1, de), jnp.float32)],
        compiler_params=pltpu.CompilerParams(
            dimension_semantics=("parallel",)),
    )(pos, ssb, h, eps, g100, partials)
    return out
